# baseline (speedup 1.0000x reference)
import functools

import jax
import jax.numpy as jnp
from jax import lax
from jax.experimental import pallas as pl
from jax.experimental.pallas import tpu as pltpu
from jax.experimental.pallas import tpu_sc as plsc

C_IN = 128
C_OUT = 128
HE = 160000
KW = 5

NC, NS = 2, 16
NW = NC * NS
CHUNK = 128
HE_PAD = 163840
EPW = HE_PAD // NW
NCH = EPW // CHUNK
NSLOT = 4

E_BLK = 3200


def _p1_body(f_ref, w_ref, b_ref, y0_ref, z1_ref, z2_ref, z3_ref, z4_ref):
    f = f_ref[...]
    y0_ref[...] = lax.dot_general(
        w_ref[0], f, (((1,), (0,)), ((), ())),
        preferred_element_type=jnp.float32) + b_ref[...]
    for k, z_ref in ((1, z1_ref), (2, z2_ref), (3, z3_ref), (4, z4_ref)):
        z_ref[...] = lax.dot_general(
            f, w_ref[k], (((0,), (1,)), ((), ())),
            preferred_element_type=jnp.float32)


def _pass1(f2d, w, b2):
    zt_shape = jax.ShapeDtypeStruct((HE, C_OUT), jnp.float32)
    return pl.pallas_call(
        _p1_body,
        grid=(HE // E_BLK,),
        in_specs=[
            pl.BlockSpec((C_IN, E_BLK), lambda i: (0, i)),
            pl.BlockSpec((KW, C_OUT, C_IN), lambda i: (0, 0, 0)),
            pl.BlockSpec((C_OUT, 1), lambda i: (0, 0)),
        ],
        out_specs=[
            pl.BlockSpec((C_OUT, E_BLK), lambda i: (0, i)),
            pl.BlockSpec((E_BLK, C_OUT), lambda i: (i, 0)),
            pl.BlockSpec((E_BLK, C_OUT), lambda i: (i, 0)),
            pl.BlockSpec((E_BLK, C_OUT), lambda i: (i, 0)),
            pl.BlockSpec((E_BLK, C_OUT), lambda i: (i, 0)),
        ],
        out_shape=[jax.ShapeDtypeStruct((C_OUT, HE), jnp.float32),
                   zt_shape, zt_shape, zt_shape, zt_shape],
    )(f2d, w, b2)


def _sc_body(z1, z2, z3, z4, nbh4, s_out,
             i0, i1, i2, i3, acc0, acc1, acc2, acc3,
             sem0, sem1, sem2, sem3):
    wid = lax.axis_index("s") * NC + lax.axis_index("c")
    base = wid * EPW
    zs = (z1, z2, z3, z4)
    idxs = (i0, i1, i2, i3)
    accs = (acc0, acc1, acc2, acc3)
    sems = (sem0, sem1, sem2, sem3)

    for k in range(4):
        pltpu.sync_copy(nbh4.at[k, wid], idxs[k])

    def fire_k0(c, u):
        pltpu.make_async_copy(zs[0].at[idxs[0].at[c]], accs[u], sems[u]).start()

    def wait_k0(u):
        pltpu.make_async_copy(zs[0].at[idxs[0].at[0]], accs[u], sems[u]).wait()

    def fire_adds(c, u):
        for k in (1, 2, 3):
            pltpu.async_copy(zs[k].at[idxs[k].at[c]], accs[u], sems[u],
                             add=True)

    def wait_adds(u):
        for _ in range(3):
            pltpu.make_async_copy(zs[0].at[idxs[0].at[0]], accs[u],
                                  sems[u]).wait()

    def fire_store(c, u):
        off = base + c * CHUNK
        pltpu.make_async_copy(accs[u], s_out.at[pl.ds(off, CHUNK)],
                              sems[u]).start()

    def wait_store(u):
        pltpu.make_async_copy(accs[u], s_out.at[pl.ds(base, CHUNK)],
                              sems[u]).wait()

    def loop_body(p, _):
        for u in range(NSLOT):
            c = NSLOT * p + u
            @pl.when(p >= 1)
            def _():
                wait_store(u)
            fire_k0(c, u)
            up = (u + NSLOT - 1) % NSLOT
            if u == 0:
                @pl.when(p >= 1)
                def _():
                    wait_k0(up)
                    fire_adds(c - 1, up)
            else:
                wait_k0(up)
                fire_adds(c - 1, up)
            um = (u + NSLOT - 2) % NSLOT
            if u >= 2:
                wait_adds(um)
                fire_store(c - 2, um)
            else:
                @pl.when(p >= 1)
                def _():
                    wait_adds(um)
                    fire_store(c - 2, um)
        return 0

    lax.fori_loop(0, NCH // NSLOT, loop_body, 0)
    wait_k0(NSLOT - 1)
    fire_adds(NCH - 1, NSLOT - 1)
    wait_adds(NSLOT - 2)
    fire_store(NCH - 2, NSLOT - 2)
    wait_adds(NSLOT - 1)
    fire_store(NCH - 1, NSLOT - 1)
    for u in range(NSLOT):
        wait_store(u)


_ACC = pltpu.VMEM((CHUNK, C_OUT), jnp.float32)
_sc_gather_sum = functools.partial(
    pl.kernel,
    out_type=jax.ShapeDtypeStruct((HE_PAD, C_OUT), jnp.float32),
    mesh=plsc.VectorSubcoreMesh(core_axis_name="c", subcore_axis_name="s"),
    scratch_types=[
        pltpu.VMEM((NCH, CHUNK), jnp.int32),
        pltpu.VMEM((NCH, CHUNK), jnp.int32),
        pltpu.VMEM((NCH, CHUNK), jnp.int32),
        pltpu.VMEM((NCH, CHUNK), jnp.int32),
        _ACC, _ACC, _ACC, _ACC,
        pltpu.SemaphoreType.DMA,
        pltpu.SemaphoreType.DMA,
        pltpu.SemaphoreType.DMA,
        pltpu.SemaphoreType.DMA,
    ],
)(_sc_body)


def _p3_body(y0_ref, s_ref, o_ref):
    o_ref[...] = y0_ref[...] + s_ref[...].T


def _pass3(y0, s):
    return pl.pallas_call(
        _p3_body,
        grid=(HE // E_BLK,),
        in_specs=[
            pl.BlockSpec((C_OUT, E_BLK), lambda i: (0, i)),
            pl.BlockSpec((E_BLK, C_OUT), lambda i: (i, 0)),
        ],
        out_specs=pl.BlockSpec((C_OUT, E_BLK), lambda i: (0, i)),
        out_shape=jax.ShapeDtypeStruct((C_OUT, HE), jnp.float32),
    )(y0, s)


def kernel(half_edge_features, neighborhoods, conv_w, conv_b):
    f2d = half_edge_features[0]
    w = jnp.transpose(conv_w[:, :, 0, :], (2, 0, 1))
    b2 = conv_b[:, None]

    y0, z1, z2, z3, z4 = _pass1(f2d, w, b2)

    nbh_t = jnp.transpose(neighborhoods[0])
    nbh4 = jnp.pad(nbh_t, ((0, 0), (0, HE_PAD - HE))).reshape(
        KW - 1, NW, NCH, CHUNK)

    s = _sc_gather_sum(z1, z2, z3, z4, nbh4)

    out = _pass3(y0, s)
    return out[None, :, :, None]
